# baseline (speedup 1.0000x reference)
import functools

import jax
import jax.numpy as jnp
from jax import lax
from jax.experimental import pallas as pl
from jax.experimental.pallas import tpu as pltpu
from jax.experimental.pallas import tpu_sc as plsc

N = 10000
E = 320000
D_IN = 128
H = 64
G = 16
B = 16

NC = 2
NS = 16
NW = NC * NS
CHUNK = 128
KG = 8
CPW = 80
EPAD = NW * CPW * CHUNK
GROUPS = CPW // KG
RPS = 632
ACC_ROWS = NS * RPS

_P = lax.Precision.HIGHEST
_F = jnp.float32

_sc_mesh = plsc.VectorSubcoreMesh(core_axis_name="c", subcore_axis_name="s")


@functools.partial(
    pl.kernel,
    out_type=jax.ShapeDtypeStruct((NC, ACC_ROWS, H), _F),
    mesh=_sc_mesh,
    scratch_types=[
        pltpu.VMEM((KG, CHUNK), jnp.int32),
        pltpu.VMEM((KG, CHUNK), jnp.int32),
        [pltpu.VMEM((CHUNK, H), _F) for _ in range(KG)],
        pltpu.VMEM_SHARED((ACC_ROWS, H), _F),
        pltpu.SemaphoreType.DMA,
        pltpu.SemaphoreType.DMA,
    ],
    compiler_params=pltpu.CompilerParams(use_tc_tiling_on_sc=False),
)
def _sc_segsum(y_hbm, src_hbm, dst_hbm, zero_hbm, out_hbm,
               sidx, didx, rows, acc, gsem, ssem):
    cid = lax.axis_index("c")
    sid = lax.axis_index("s")
    wid = sid * NC + cid
    pltpu.sync_copy(zero_hbm, acc.at[pl.ds(sid * RPS, RPS)])
    plsc.subcore_barrier()

    def body(grp, carry):
        base = wid * CPW + grp * KG
        pltpu.sync_copy(src_hbm.at[pl.ds(base, KG)], sidx)
        pltpu.sync_copy(dst_hbm.at[pl.ds(base, KG)], didx)
        gathers = [pltpu.async_copy(y_hbm.at[sidx.at[j]], rows[j], gsem)
                   for j in range(KG)]
        for g in gathers:
            g.wait()
        for j in range(KG):
            pltpu.sync_copy(rows[j], acc.at[didx.at[j]], add=True)
        return carry

    lax.fori_loop(0, GROUPS, body, 0)
    plsc.subcore_barrier()
    pltpu.sync_copy(acc.at[pl.ds(sid * RPS, RPS)],
                    out_hbm.at[cid, pl.ds(sid * RPS, RPS)])


def _mm_body(x_ref, w_ref, o_ref):
    o_ref[...] = jnp.dot(x_ref[...], w_ref[...],
                         preferred_element_type=_F, precision=_P)


def _mlp_body(y_ref, p_ref, b1_ref, w2_ref, b2_ref, w3_ref, o_ref):
    u = jnp.maximum(y_ref[...] + p_ref[0, :N] + p_ref[1, :N] + b1_ref[...], 0.0)
    h = jnp.maximum(
        jnp.dot(u, w2_ref[...], preferred_element_type=_F, precision=_P)
        + b2_ref[...], 0.0)
    o_ref[...] = jnp.dot(h, w3_ref[...], preferred_element_type=_F, precision=_P)


def _final_body(y_ref, q_ref, b3_ref, w4_ref, b4_ref, bidx_ref,
                wih_t_ref, bih_ref, bhh_ref, wout_t_ref, bout_ref,
                len_ref, o_ref):
    u = jnp.maximum(y_ref[...] + q_ref[0, :N] + q_ref[1, :N] + b3_ref[...], 0.0)
    h2 = jnp.maximum(
        jnp.dot(u, w4_ref[...], preferred_element_type=_F, precision=_P)
        + b4_ref[...], 0.0)
    seg = lax.broadcasted_iota(jnp.int32, (G, N), 0)
    onehot = (seg == jnp.broadcast_to(bidx_ref[...], (G, N))).astype(_F)
    sums = jnp.dot(onehot, h2, preferred_element_type=_F, precision=_P)
    counts = jnp.sum(onehot, axis=1, keepdims=True)
    g = sums / jnp.maximum(counts, 1.0)
    gi = jnp.dot(g, wih_t_ref[...], preferred_element_type=_F,
                 precision=_P) + bih_ref[...]
    gh = jnp.broadcast_to(bhh_ref[...], (B, 3 * H))
    r = jax.nn.sigmoid(gi[:, :H] + gh[:, :H])
    zz = jax.nn.sigmoid(gi[:, H:2 * H] + gh[:, H:2 * H])
    nn_ = jnp.tanh(gi[:, 2 * H:] + r * gh[:, 2 * H:])
    h_new = (1.0 - zz) * nn_
    m = jnp.reshape(len_ref[...], (B, 1)) > 0
    h_last = jnp.where(m, h_new, 0.0)
    o_ref[...] = jnp.dot(h_last, wout_t_ref[...],
                         preferred_element_type=_F, precision=_P) + bout_ref[...]


def kernel(x, edge_index, batch_idx, lengths, W1, b1, W2, b2, W3, b3, W4, b4,
           W_ih, b_ih, W_hh, b_hh, W_out, b_out):
    pad = EPAD - E
    src = jnp.concatenate([edge_index[0], jnp.zeros((pad,), jnp.int32)])
    dst = jnp.concatenate([edge_index[1], jnp.full((pad,), N, jnp.int32)])
    src2 = src.reshape(NW * CPW, CHUNK)
    dst2 = dst.reshape(NW * CPW, CHUNK)
    zeros = jnp.zeros((RPS, H), _F)

    y1 = pl.pallas_call(
        _mm_body, out_shape=jax.ShapeDtypeStruct((N, H), _F))(x, W1)

    p = _sc_segsum(y1, src2, dst2, zeros)

    y2 = pl.pallas_call(
        _mlp_body, out_shape=jax.ShapeDtypeStruct((N, H), _F))(
            y1, p, b1.reshape(1, H), W2, b2.reshape(1, H), W3)

    q = _sc_segsum(y2, src2, dst2, zeros)

    out = pl.pallas_call(
        _final_body, out_shape=jax.ShapeDtypeStruct((B, 1), _F))(
            y2, q, b3.reshape(1, H), W4, b4.reshape(1, H),
            batch_idx.reshape(1, N), W_ih.T, b_ih.reshape(1, 3 * H),
            b_hh.reshape(1, 3 * H), W_out.T, b_out.reshape(1, 1),
            lengths.reshape(1, B))
    return out[:, 0]


# speedup vs baseline: 1.0476x; 1.0476x over previous
import functools

import jax
import jax.numpy as jnp
from jax import lax
from jax.experimental import pallas as pl
from jax.experimental.pallas import tpu as pltpu
from jax.experimental.pallas import tpu_sc as plsc

N = 10000
E = 320000
D_IN = 128
H = 64
G = 16
B = 16

NC = 2
NS = 16
NW = NC * NS
CHUNK = 128
KG = 5
CPW = 80
EPAD = NW * CPW * CHUNK
GROUPS = CPW // KG
PAIRS = GROUPS // 2
RPS = 632
ACC_ROWS = NS * RPS

_P = lax.Precision.HIGHEST
_F = jnp.float32

_sc_mesh = plsc.VectorSubcoreMesh(core_axis_name="c", subcore_axis_name="s")


@functools.partial(
    pl.kernel,
    out_type=jax.ShapeDtypeStruct((NC, ACC_ROWS, H), _F),
    mesh=_sc_mesh,
    scratch_types=[
        [pltpu.VMEM((KG, CHUNK), jnp.int32) for _ in range(4)],
        [pltpu.VMEM((CHUNK, H), _F) for _ in range(2 * KG)],
        pltpu.VMEM_SHARED((ACC_ROWS, H), _F),
        [pltpu.SemaphoreType.DMA for _ in range(2 * KG)],
    ],
    compiler_params=pltpu.CompilerParams(use_tc_tiling_on_sc=False),
)
def _sc_segsum(y_hbm, src_hbm, dst_hbm, zero_hbm, out_hbm,
               idx, rows, acc, sems):
    sidxA, didxA, sidxB, didxB = idx
    cid = lax.axis_index("c")
    sid = lax.axis_index("s")
    wid = sid * NC + cid
    pltpu.sync_copy(zero_hbm, acc.at[pl.ds(sid * RPS, RPS)])
    plsc.subcore_barrier()

    def _load_idx(grp, sbuf, dbuf):
        base = wid * CPW + grp * KG
        pltpu.sync_copy(src_hbm.at[pl.ds(base, KG)], sbuf)
        pltpu.sync_copy(dst_hbm.at[pl.ds(base, KG)], dbuf)

    def _fire(sbuf, half):
        for j in range(KG):
            pltpu.async_copy(y_hbm.at[sbuf.at[j]], rows[half * KG + j],
                             sems[half * KG + j])

    def _drain_scatter(dbuf, half, scatter):
        for j in range(KG):
            pltpu.make_async_copy(y_hbm.at[sbuf0.at[j]], rows[half * KG + j],
                                  sems[half * KG + j]).wait()
            if scatter:
                pltpu.sync_copy(rows[half * KG + j], acc.at[dbuf.at[j]],
                                add=True)

    sbuf0 = sidxA

    _load_idx(0, sidxA, didxA)
    _fire(sidxA, 0)

    def body(i, carry):
        g = 2 * i
        _load_idx(g + 1, sidxB, didxB)
        _fire(sidxB, 1)
        _drain_scatter(didxA, 0, True)
        _load_idx(jnp.minimum(g + 2, GROUPS - 1), sidxA, didxA)
        _fire(sidxA, 0)
        _drain_scatter(didxB, 1, True)
        return carry

    lax.fori_loop(0, PAIRS, body, 0)
    _drain_scatter(didxA, 0, False)
    plsc.subcore_barrier()
    pltpu.sync_copy(acc.at[pl.ds(sid * RPS, RPS)],
                    out_hbm.at[cid, pl.ds(sid * RPS, RPS)])


def _mm_body(x_ref, w_ref, o_ref):
    o_ref[...] = jnp.dot(x_ref[...], w_ref[...],
                         preferred_element_type=_F, precision=_P)


def _mlp_body(y_ref, p_ref, b1_ref, w2_ref, b2_ref, w3_ref, o_ref):
    u = jnp.maximum(y_ref[...] + p_ref[0, :N] + p_ref[1, :N] + b1_ref[...], 0.0)
    h = jnp.maximum(
        jnp.dot(u, w2_ref[...], preferred_element_type=_F, precision=_P)
        + b2_ref[...], 0.0)
    o_ref[...] = jnp.dot(h, w3_ref[...], preferred_element_type=_F, precision=_P)


def _final_body(y_ref, q_ref, b3_ref, w4_ref, b4_ref, bidx_ref,
                wih_t_ref, bih_ref, bhh_ref, wout_t_ref, bout_ref,
                len_ref, o_ref):
    u = jnp.maximum(y_ref[...] + q_ref[0, :N] + q_ref[1, :N] + b3_ref[...], 0.0)
    h2 = jnp.maximum(
        jnp.dot(u, w4_ref[...], preferred_element_type=_F, precision=_P)
        + b4_ref[...], 0.0)
    seg = lax.broadcasted_iota(jnp.int32, (G, N), 0)
    onehot = (seg == jnp.broadcast_to(bidx_ref[...], (G, N))).astype(_F)
    sums = jnp.dot(onehot, h2, preferred_element_type=_F, precision=_P)
    counts = jnp.sum(onehot, axis=1, keepdims=True)
    g = sums / jnp.maximum(counts, 1.0)
    gi = jnp.dot(g, wih_t_ref[...], preferred_element_type=_F,
                 precision=_P) + bih_ref[...]
    gh = jnp.broadcast_to(bhh_ref[...], (B, 3 * H))
    r = jax.nn.sigmoid(gi[:, :H] + gh[:, :H])
    zz = jax.nn.sigmoid(gi[:, H:2 * H] + gh[:, H:2 * H])
    nn_ = jnp.tanh(gi[:, 2 * H:] + r * gh[:, 2 * H:])
    h_new = (1.0 - zz) * nn_
    m = jnp.reshape(len_ref[...], (B, 1)) > 0
    h_last = jnp.where(m, h_new, 0.0)
    o_ref[...] = jnp.dot(h_last, wout_t_ref[...],
                         preferred_element_type=_F, precision=_P) + bout_ref[...]


def kernel(x, edge_index, batch_idx, lengths, W1, b1, W2, b2, W3, b3, W4, b4,
           W_ih, b_ih, W_hh, b_hh, W_out, b_out):
    pad = EPAD - E
    src = jnp.concatenate([edge_index[0], jnp.zeros((pad,), jnp.int32)])
    dst = jnp.concatenate([edge_index[1], jnp.full((pad,), N, jnp.int32)])
    src2 = src.reshape(NW * CPW, CHUNK)
    dst2 = dst.reshape(NW * CPW, CHUNK)
    zeros = jnp.zeros((RPS, H), _F)

    y1 = pl.pallas_call(
        _mm_body, out_shape=jax.ShapeDtypeStruct((N, H), _F))(x, W1)

    p = _sc_segsum(y1, src2, dst2, zeros)

    y2 = pl.pallas_call(
        _mlp_body, out_shape=jax.ShapeDtypeStruct((N, H), _F))(
            y1, p, b1.reshape(1, H), W2, b2.reshape(1, H), W3)

    q = _sc_segsum(y2, src2, dst2, zeros)

    out = pl.pallas_call(
        _final_body, out_shape=jax.ShapeDtypeStruct((B, 1), _F))(
            y2, q, b3.reshape(1, H), W4, b4.reshape(1, H),
            batch_idx.reshape(1, N), W_ih.T, b_ih.reshape(1, 3 * H),
            b_hh.reshape(1, 3 * H), W_out.T, b_out.reshape(1, 1),
            lengths.reshape(1, B))
    return out[:, 0]
